# baseline (speedup 1.0000x reference)
import functools

import jax
import jax.numpy as jnp
from jax import lax
from jax.experimental import pallas as pl
from jax.experimental.pallas import tpu as pltpu
from jax.experimental.pallas import tpu_sc as plsc

B, T = 4, 8192
VOCAB, CHAR_DIM, N_FREQ = 1000, 32, 4
OUT_DIM = CHAR_DIM * N_FREQ
NTOK = B * T
NUM_CORES, NUM_SUBCORES, LANES = 2, 16, 16
NW = NUM_CORES * NUM_SUBCORES
TPW = NTOK // NW
CHUNK = 128
TPOS = T // NW
NTC = TPOS // CHUNK
NU = NTC * B
RPB = 128
NBLK = VOCAB // RPB
NTAIL = VOCAB - NBLK * RPB

_mesh = plsc.VectorSubcoreMesh(core_axis_name="c", subcore_axis_name="s")


@functools.partial(
    pl.kernel,
    out_type=jax.ShapeDtypeStruct((NTOK, OUT_DIM), jnp.float32),
    mesh=_mesh,
    scratch_types=[
        [pltpu.VMEM((CHUNK,), jnp.int32) for _ in range(NU)],
        [pltpu.VMEM((CHUNK, OUT_DIM), jnp.float32) for _ in range(2)],
        [pltpu.VMEM((CHUNK, OUT_DIM), jnp.float32) for _ in range(3)],
        pltpu.VMEM_SHARED((VOCAB, OUT_DIM), jnp.float32),
        [pltpu.SemaphoreType.DMA for _ in range(NU)],
        [pltpu.SemaphoreType.DMA for _ in range(2)],
        [pltpu.SemaphoreType.DMA for _ in range(3)],
        [pltpu.SemaphoreType.DMA for _ in range(3)],
    ],
)
def _encode(idx_hbm, cat_hbm, pos_hbm, out_hbm,
            idx_v, pos_v, rows_v, cat_sh, si, sp, sg, so):
    sid = lax.axis_index("s")
    w = sid * NUM_CORES + lax.axis_index("c")
    t_base = w * TPOS

    r0 = jnp.minimum(sid * 64, VOCAB - 64)
    pltpu.sync_copy(cat_hbm.at[pl.ds(r0, 64)], cat_sh.at[pl.ds(r0, 64)])
    plsc.subcore_barrier()

    def bt0_of(u):
        tc, b = divmod(u, B)
        return b, t_base + tc * CHUNK

    def tok0_of(u):
        b, t0 = bt0_of(u)
        return b * T + t0

    def start_idx(u):
        b, t0 = bt0_of(u)
        return pltpu.async_copy(
            idx_hbm.at[b, pl.ds(t0, CHUNK)], idx_v[u], si[u])

    def start_pos(tc):
        return pltpu.async_copy(
            pos_hbm.at[pl.ds(t_base + tc * CHUNK, CHUNK)], pos_v[tc], sp[tc])

    def start_gather(u):
        p = u % 3
        return pltpu.async_copy(cat_sh.at[idx_v[u]], rows_v[p], sg[p])

    def start_out_half(u, h):
        p = u % 3
        return pltpu.async_copy(
            rows_v[p].at[pl.ds(h * (CHUNK // 2), CHUNK // 2)],
            out_hbm.at[pl.ds(tok0_of(u) + h * (CHUNK // 2), CHUNK // 2)],
            so[p])

    di = []
    for u in range(NU):
        di.append(start_idx(u))
        if u == 0:
            dpos = [start_pos(0), start_pos(1)]
    di[0].wait()
    di[1].wait()
    dg = [start_gather(0), start_gather(1), None]
    dout = [None, None, None]
    dpos[0].wait()
    dpos[1].wait()

    for u in range(NU):
        p = u % 3
        if u + 2 < NU:
            q = (u + 2) % 3
            if dout[q] is not None:
                dout[q][0].wait()
                dout[q][1].wait()
                dout[q] = None
            di[u + 2].wait()
            dg[q] = start_gather(u + 2)
        dg[p].wait()

        rows = rows_v[p]
        pos = pos_v[u // B]

        def tok_body(i4, c2, rows=rows, pos=pos):
            for j in range(4):
                i = i4 * 4 + j
                for k in range(OUT_DIM // LANES):
                    v = pos[i, pl.ds(k * LANES, LANES)]
                    plsc.addupdate(rows.at[i, pl.ds(k * LANES, LANES)], v)
            return c2

        def tok_body_hi(i4, c2, rows=rows, pos=pos):
            return tok_body(i4 + CHUNK // 8, c2, rows=rows, pos=pos)

        lax.fori_loop(0, CHUNK // 8, tok_body, 0)
        d_lo = start_out_half(u, 0)
        lax.fori_loop(0, CHUNK // 8, tok_body_hi, 0)
        d_hi = start_out_half(u, 1)
        dout[p] = (d_lo, d_hi)

    for d in dout:
        if d is not None:
            d[0].wait()
            d[1].wait()


def kernel(raw_char_indices, emb0, emb1, emb2, emb3, pos_table):
    cat = jnp.concatenate([emb0, emb1, emb2, emb3], axis=1)
    out = _encode(raw_char_indices, cat, pos_table)
    return out.reshape(B, T, OUT_DIM)


# speedup vs baseline: 1.0060x; 1.0060x over previous
import functools

import jax
import jax.numpy as jnp
from jax import lax
from jax.experimental import pallas as pl
from jax.experimental.pallas import tpu as pltpu
from jax.experimental.pallas import tpu_sc as plsc

B, T = 4, 8192
VOCAB, CHAR_DIM, N_FREQ = 1000, 32, 4
OUT_DIM = CHAR_DIM * N_FREQ
NTOK = B * T
NUM_CORES, NUM_SUBCORES, LANES = 2, 16, 16
NW = NUM_CORES * NUM_SUBCORES
TPW = NTOK // NW
CHUNK = 128
TPOS = T // NW
NTC = TPOS // CHUNK
NU = NTC * B
RPB = 128
NBLK = VOCAB // RPB
NTAIL = VOCAB - NBLK * RPB

_mesh = plsc.VectorSubcoreMesh(core_axis_name="c", subcore_axis_name="s")


@functools.partial(
    pl.kernel,
    out_type=jax.ShapeDtypeStruct((NTOK, OUT_DIM), jnp.float32),
    mesh=_mesh,
    scratch_types=[
        [pltpu.VMEM((CHUNK,), jnp.int32) for _ in range(NU)],
        [pltpu.VMEM((CHUNK, OUT_DIM), jnp.float32) for _ in range(2)],
        [pltpu.VMEM((CHUNK, OUT_DIM), jnp.float32) for _ in range(3)],
        pltpu.VMEM_SHARED((VOCAB, OUT_DIM), jnp.float32),
        [pltpu.SemaphoreType.DMA for _ in range(NU)],
        [pltpu.SemaphoreType.DMA for _ in range(2)],
        [pltpu.SemaphoreType.DMA for _ in range(3)],
        [pltpu.SemaphoreType.DMA for _ in range(3)],
    ],
)
def _encode(idx_hbm, cat_hbm, pos_hbm, out_hbm,
            idx_v, pos_v, rows_v, cat_sh, si, sp, sg, so):
    sid = lax.axis_index("s")
    w = sid * NUM_CORES + lax.axis_index("c")
    t_base = w * TPOS

    r0 = jnp.minimum(sid * 64, VOCAB - 64)
    pltpu.sync_copy(cat_hbm.at[pl.ds(r0, 64)], cat_sh.at[pl.ds(r0, 64)])
    plsc.subcore_barrier()

    def bt0_of(u):
        tc, b = divmod(u, B)
        return b, t_base + tc * CHUNK

    def tok0_of(u):
        b, t0 = bt0_of(u)
        return b * T + t0

    def start_idx(u):
        b, t0 = bt0_of(u)
        return pltpu.async_copy(
            idx_hbm.at[b, pl.ds(t0, CHUNK)], idx_v[u], si[u])

    def start_pos(tc):
        return pltpu.async_copy(
            pos_hbm.at[pl.ds(t_base + tc * CHUNK, CHUNK)], pos_v[tc], sp[tc])

    def start_gather(u):
        p = u % 3
        return pltpu.async_copy(cat_sh.at[idx_v[u]], rows_v[p], sg[p])

    def start_out(u):
        p = u % 3
        return pltpu.async_copy(
            rows_v[p], out_hbm.at[pl.ds(tok0_of(u), CHUNK)], so[p])

    di = []
    for u in range(NU):
        di.append(start_idx(u))
        if u == 0:
            dpos = [start_pos(0), start_pos(1)]
    di[0].wait()
    di[1].wait()
    dg = [start_gather(0), start_gather(1), None]
    dout = [None, None, None]
    dpos[0].wait()
    dpos[1].wait()

    for u in range(NU):
        p = u % 3
        if u + 2 < NU:
            q = (u + 2) % 3
            if dout[q] is not None:
                dout[q].wait()
                dout[q] = None
            di[u + 2].wait()
            dg[q] = start_gather(u + 2)
        dg[p].wait()

        rows = rows_v[p]
        pos = pos_v[u // B]

        def tok_body(i2, c2, rows=rows, pos=pos):
            for j in range(2):
                i = i2 * 2 + j
                for k in range(OUT_DIM // LANES):
                    v = pos[i, pl.ds(k * LANES, LANES)]
                    plsc.addupdate(rows.at[i, pl.ds(k * LANES, LANES)], v)
            return c2

        lax.fori_loop(0, CHUNK // 2, tok_body, 0)
        dout[p] = start_out(u)

    for d in dout:
        if d is not None:
            d.wait()


def kernel(raw_char_indices, emb0, emb1, emb2, emb3, pos_table):
    cat = jnp.concatenate([emb0, emb1, emb2, emb3], axis=1)
    out = _encode(raw_char_indices, cat, pos_table)
    return out.reshape(B, T, OUT_DIM)
